# baseline (speedup 1.0000x reference)
import functools

import jax
import jax.numpy as jnp
from jax import lax
from jax.experimental import pallas as pl
from jax.experimental.pallas import tpu as pltpu
from jax.experimental.pallas import tpu_sc as plsc

_EMBED_DIM = 32
_FC_DIM = 200
_BATCH = 16384

_NC = 2
_NS = 16
_NW = _NC * _NS
_CHUNK = 128
_GRID_ROWS = _BATCH // _CHUNK
_ROWS_PER_W = _GRID_ROWS // _NW


def _make_gather():
    mesh = plsc.VectorSubcoreMesh(
        core_axis_name="c", subcore_axis_name="s",
        num_cores=_NC, num_subcores=_NS)

    @functools.partial(
        pl.kernel,
        mesh=mesh,
        out_type=jax.ShapeDtypeStruct((_GRID_ROWS, _CHUNK, _EMBED_DIM),
                                      jnp.float32),
        scratch_types=[
            pltpu.VMEM((_ROWS_PER_W, _CHUNK), jnp.int32),
            pltpu.VMEM((_ROWS_PER_W, _CHUNK, _EMBED_DIM), jnp.float32),
            pltpu.SemaphoreType.DMA,
        ],
        compiler_params=pltpu.CompilerParams(use_tc_tiling_on_sc=False),
    )
    def gather(idx_hbm, table_hbm, out_hbm, idx_v, rows_v, sem):
        wid = lax.axis_index("s") * _NC + lax.axis_index("c")
        base = wid * _ROWS_PER_W
        pltpu.sync_copy(idx_hbm.at[pl.ds(base, _ROWS_PER_W)], idx_v)
        copies = []
        for j in range(_ROWS_PER_W):
            copies.append(
                pltpu.async_copy(table_hbm.at[idx_v.at[j]], rows_v.at[j], sem))
        for c in copies:
            c.wait()
        pltpu.sync_copy(rows_v, out_hbm.at[pl.ds(base, _ROWS_PER_W)])

    return gather


_gather = _make_gather()


def _fc_body(emb_ref, w_ref, b_ref, out_ref):
    acc = jnp.dot(emb_ref[...], w_ref[...],
                  preferred_element_type=jnp.float32)
    out_ref[...] = jnp.maximum(acc + b_ref[...], 0.0)


def _fc(emb, W, b2d):
    blk = 2048
    return pl.pallas_call(
        _fc_body,
        grid=(_BATCH // blk,),
        in_specs=[
            pl.BlockSpec((blk, _EMBED_DIM), lambda i: (i, 0)),
            pl.BlockSpec((_EMBED_DIM, _FC_DIM), lambda i: (0, 0)),
            pl.BlockSpec((1, _FC_DIM), lambda i: (0, 0)),
        ],
        out_specs=pl.BlockSpec((blk, _FC_DIM), lambda i: (i, 0)),
        out_shape=jax.ShapeDtypeStruct((_BATCH, _FC_DIM), jnp.float32),
    )(emb, W, b2d)


def kernel(indices, table, W, b):
    idx = indices.reshape(_GRID_ROWS, _CHUNK).astype(jnp.int32)
    gathered = _gather(idx, table)
    emb = gathered.reshape(_BATCH, _EMBED_DIM)
    out = _fc(emb, W, b.reshape(1, _FC_DIM))
    return out.reshape(_BATCH, 1, _FC_DIM)
